# baseline (speedup 1.0000x reference)
import jax
import jax.numpy as jnp
from jax.experimental import pallas as pl

_N = 32768
_R = 256
_C = 128
_K = 64


def _swap(x, d, rows, mlow):
    if d >= 8:
        parts = []
        for s in range(0, rows, 2 * d):
            parts.append(x[s + d:s + 2 * d])
            parts.append(x[s:s + d])
        return jnp.concatenate(parts, axis=0)
    up = jnp.concatenate([x[d:], x[:d]], axis=0)
    dn = jnp.concatenate([x[rows - d:], x[:rows - d]], axis=0)
    return jnp.where(mlow, up, dn)


def _gt(va, ia, vb, ib):
    return (va > vb) | ((va == vb) & (ia < ib))


def _body(x_ref, out_ref):
    v = x_ref[:]
    i = (jax.lax.broadcasted_iota(jnp.int32, (_R, _C), 0) * _C
         + jax.lax.broadcasted_iota(jnp.int32, (_R, _C), 1))
    riota = jax.lax.broadcasted_iota(jnp.int32, (_R, _C), 0)
    mlow = {d: (riota & d) == 0 for d in (1, 2, 4, 8, 16, 32, 64, 128)}

    k = 2
    while k <= _R:
        mk = mlow[k] if k <= 128 else None
        d = k // 2
        while d >= 1:
            vp = _swap(v, d, _R, mlow[d])
            ip = _swap(i, d, _R, mlow[d])
            ge = _gt(v, i, vp, ip)
            take_max = mlow[d] if mk is None else (mlow[d] == mk)
            takex = ge == take_max
            v = jnp.where(takex, v, vp)
            i = jnp.where(takex, i, ip)
            d //= 2
        k *= 2

    y, yi = v[:_K], i[:_K]

    r64 = jax.lax.broadcasted_iota(jnp.int32, (_K, _C), 0)
    m64 = {d: (r64 & d) == 0 for d in (1, 2, 4, 8, 16, 32)}
    w = _C // 2
    while w >= 1:
        a, ai = y[:, :w], yi[:, :w]
        b, bi = y[:, w:2 * w], yi[:, w:2 * w]
        for d in (32, 16, 8, 4, 2, 1):
            md = m64[d][:, :w]
            b = _swap(b, d, _K, md)
            bi = _swap(bi, d, _K, md)
        ge = _gt(a, ai, b, bi)
        y = jnp.where(ge, a, b)
        yi = jnp.where(ge, ai, bi)
        for d in (32, 16, 8, 4, 2, 1):
            md = m64[d][:, :w]
            vp = _swap(y, d, _K, md)
            ip = _swap(yi, d, _K, md)
            ge = _gt(y, yi, vp, ip)
            takex = ge == md
            y = jnp.where(takex, y, vp)
            yi = jnp.where(takex, yi, ip)
        w //= 2

    r6 = jax.lax.broadcasted_iota(jnp.int32, (_K, _K), 0)
    c6 = jax.lax.broadcasted_iota(jnp.int32, (_K, _K), 1)
    vb = jnp.broadcast_to(y, (_K, _K))
    ib = jnp.broadcast_to(yi, (_K, _K))
    vals = jnp.sum(jnp.where(r6 == c6, vb, 0.0), axis=0, keepdims=True)
    idxs = jnp.sum(jnp.where(r6 == c6, ib, 0), axis=0, keepdims=True)
    cs = jnp.sum(jnp.where(r6 <= c6, vb, 0.0), axis=0, keepdims=True)

    j64 = jax.lax.broadcasted_iota(jnp.int32, (1, _K), 1)
    kvec = (j64 + 1).astype(jnp.float32)
    support = (vals - (cs - 1.0) / kvec) > 0.0
    kz = jnp.sum(support.astype(jnp.int32), axis=(0, 1), keepdims=True)
    cs_at = jnp.sum(jnp.where(j64 == kz - 1, cs, 0.0),
                    axis=(0, 1), keepdims=True)
    tau = (cs_at - 1.0) / kz.astype(jnp.float32)

    row0 = x_ref[0:1, :]
    avail = row0 <= tau
    rc = jax.lax.broadcasted_iota(jnp.int32, (_C, _C), 0)
    cc = jax.lax.broadcasted_iota(jnp.int32, (_C, _C), 1)
    ab = jnp.broadcast_to(avail, (_C, _C))
    pc_col = jnp.sum(jnp.where(ab & (cc <= rc), 1, 0), axis=1, keepdims=True)
    av_col = jnp.sum(jnp.where(ab & (cc == rc), 1, 0), axis=1, keepdims=True)
    tgt_col = kz + pc_col - 1
    mfill = (av_col > 0) & (tgt_col == cc)
    fill = jnp.sum(jnp.where(mfill, rc, 0), axis=0, keepdims=True)

    jcol = jax.lax.broadcasted_iota(jnp.int32, (1, _C), 1)
    idx128 = jnp.concatenate([idxs, jnp.zeros((1, _C - _K), jnp.int32)],
                             axis=1)
    out = jnp.where(jcol < kz, idx128, fill)
    out_ref[:] = jnp.broadcast_to(out, (8, _C))


def kernel(scores):
    x = scores.reshape(_R, _C)
    out = pl.pallas_call(
        _body,
        out_shape=jax.ShapeDtypeStruct((8, _C), jnp.int32),
    )(x)
    return out[0, :_K]


# speedup vs baseline: 1.0720x; 1.0720x over previous
import jax
import jax.numpy as jnp
from jax.experimental import pallas as pl

_N = 32768
_R = 256
_C = 128
_K = 64


def _swap(x, d, rows, mlow):
    if d >= 8:
        parts = []
        for s in range(0, rows, 2 * d):
            parts.append(x[s + d:s + 2 * d])
            parts.append(x[s:s + d])
        return jnp.concatenate(parts, axis=0)
    up = jnp.concatenate([x[d:], x[:d]], axis=0)
    dn = jnp.concatenate([x[rows - d:], x[:rows - d]], axis=0)
    return jnp.where(mlow, up, dn)


def _gt(va, ia, vb, ib):
    return (va > vb) | ((va == vb) & (ia < ib))


def _body(x_ref, out_ref):
    v = x_ref[:]
    i = (jax.lax.broadcasted_iota(jnp.int32, (_R, _C), 0) * _C
         + jax.lax.broadcasted_iota(jnp.int32, (_R, _C), 1))
    riota = jax.lax.broadcasted_iota(jnp.int32, (_R, _C), 0)
    mlow = {d: (riota & d) == 0 for d in (1, 2, 4, 8, 16, 32, 64, 128)}

    k = 2
    while k <= _K:
        mk = mlow[k]
        d = k // 2
        while d >= 1:
            vp = _swap(v, d, _R, mlow[d])
            ip = _swap(i, d, _R, mlow[d])
            ge = _gt(v, i, vp, ip)
            take_max = mlow[d] == mk
            takex = ge == take_max
            v = jnp.where(takex, v, vp)
            i = jnp.where(takex, i, ip)
            d //= 2
        k *= 2

    ge = _gt(v[:_K], i[:_K], v[_K:2 * _K], i[_K:2 * _K])
    y1 = jnp.where(ge, v[:_K], v[_K:2 * _K])
    y1i = jnp.where(ge, i[:_K], i[_K:2 * _K])
    ge = _gt(v[2 * _K:3 * _K], i[2 * _K:3 * _K], v[3 * _K:], i[3 * _K:])
    y2 = jnp.where(ge, v[2 * _K:3 * _K], v[3 * _K:])
    y2i = jnp.where(ge, i[2 * _K:3 * _K], i[3 * _K:])
    z = jnp.concatenate([y1, y2], axis=0)
    zi = jnp.concatenate([y1i, y2i], axis=0)
    mblk = mlow[_K][:2 * _K]
    for d in (32, 16, 8, 4, 2, 1):
        md = mlow[d][:2 * _K]
        vp = _swap(z, d, 2 * _K, md)
        ip = _swap(zi, d, 2 * _K, md)
        ge = _gt(z, zi, vp, ip)
        takex = ge == (md == mblk)
        z = jnp.where(takex, z, vp)
        zi = jnp.where(takex, zi, ip)
    ge = _gt(z[:_K], zi[:_K], z[_K:], zi[_K:])
    y = jnp.where(ge, z[:_K], z[_K:])
    yi = jnp.where(ge, zi[:_K], zi[_K:])
    for d in (32, 16, 8, 4, 2, 1):
        md = mlow[d][:_K]
        vp = _swap(y, d, _K, md)
        ip = _swap(yi, d, _K, md)
        ge = _gt(y, yi, vp, ip)
        takex = ge == md
        y = jnp.where(takex, y, vp)
        yi = jnp.where(takex, yi, ip)

    r64 = jax.lax.broadcasted_iota(jnp.int32, (_K, _C), 0)
    m64 = {d: (r64 & d) == 0 for d in (1, 2, 4, 8, 16, 32)}
    w = _C // 2
    while w >= 1:
        a, ai = y[:, :w], yi[:, :w]
        b, bi = y[:, w:2 * w], yi[:, w:2 * w]
        for d in (32, 16, 8, 4, 2, 1):
            md = m64[d][:, :w]
            b = _swap(b, d, _K, md)
            bi = _swap(bi, d, _K, md)
        ge = _gt(a, ai, b, bi)
        y = jnp.where(ge, a, b)
        yi = jnp.where(ge, ai, bi)
        for d in (32, 16, 8, 4, 2, 1):
            md = m64[d][:, :w]
            vp = _swap(y, d, _K, md)
            ip = _swap(yi, d, _K, md)
            ge = _gt(y, yi, vp, ip)
            takex = ge == md
            y = jnp.where(takex, y, vp)
            yi = jnp.where(takex, yi, ip)
        w //= 2

    r6 = jax.lax.broadcasted_iota(jnp.int32, (_K, _K), 0)
    c6 = jax.lax.broadcasted_iota(jnp.int32, (_K, _K), 1)
    vb = jnp.broadcast_to(y, (_K, _K))
    ib = jnp.broadcast_to(yi, (_K, _K))
    vals = jnp.sum(jnp.where(r6 == c6, vb, 0.0), axis=0, keepdims=True)
    idxs = jnp.sum(jnp.where(r6 == c6, ib, 0), axis=0, keepdims=True)
    cs = jnp.sum(jnp.where(r6 <= c6, vb, 0.0), axis=0, keepdims=True)

    j64 = jax.lax.broadcasted_iota(jnp.int32, (1, _K), 1)
    kvec = (j64 + 1).astype(jnp.float32)
    support = (vals - (cs - 1.0) / kvec) > 0.0
    kz = jnp.sum(support.astype(jnp.int32), axis=(0, 1), keepdims=True)
    cs_at = jnp.sum(jnp.where(j64 == kz - 1, cs, 0.0),
                    axis=(0, 1), keepdims=True)
    tau = (cs_at - 1.0) / kz.astype(jnp.float32)

    row0 = x_ref[0:1, :]
    avail = row0 <= tau
    rc = jax.lax.broadcasted_iota(jnp.int32, (_C, _C), 0)
    cc = jax.lax.broadcasted_iota(jnp.int32, (_C, _C), 1)
    ab = jnp.broadcast_to(avail, (_C, _C))
    pc_col = jnp.sum(jnp.where(ab & (cc <= rc), 1, 0), axis=1, keepdims=True)
    av_col = jnp.sum(jnp.where(ab & (cc == rc), 1, 0), axis=1, keepdims=True)
    tgt_col = kz + pc_col - 1
    mfill = (av_col > 0) & (tgt_col == cc)
    fill = jnp.sum(jnp.where(mfill, rc, 0), axis=0, keepdims=True)

    jcol = jax.lax.broadcasted_iota(jnp.int32, (1, _C), 1)
    idx128 = jnp.concatenate([idxs, jnp.zeros((1, _C - _K), jnp.int32)],
                             axis=1)
    out = jnp.where(jcol < kz, idx128, fill)
    out_ref[:] = jnp.broadcast_to(out, (8, _C))


def kernel(scores):
    x = scores.reshape(_R, _C)
    out = pl.pallas_call(
        _body,
        out_shape=jax.ShapeDtypeStruct((8, _C), jnp.int32),
    )(x)
    return out[0, :_K]


# speedup vs baseline: 1.2772x; 1.1914x over previous
import jax
import jax.numpy as jnp
from jax.experimental import pallas as pl

_N = 32768
_R = 256
_C = 128
_K = 64


def _swap(x, d, rows, mlow):
    if d >= 8:
        parts = []
        for s in range(0, rows, 2 * d):
            parts.append(x[s + d:s + 2 * d])
            parts.append(x[s:s + d])
        return jnp.concatenate(parts, axis=0)
    up = jnp.concatenate([x[d:], x[:d]], axis=0)
    dn = jnp.concatenate([x[rows - d:], x[:rows - d]], axis=0)
    return jnp.where(mlow, up, dn)


def _gt(va, ia, vb, ib):
    return (va > vb) | ((va == vb) & (ia < ib))


def _body(x_ref, out_ref):
    v = x_ref[:]
    i = (jax.lax.broadcasted_iota(jnp.int32, (_R, _C), 0) * _C
         + jax.lax.broadcasted_iota(jnp.int32, (_R, _C), 1))
    riota = jax.lax.broadcasted_iota(jnp.int32, (_R, _C), 0)
    mlow = {d: (riota & d) == 0 for d in (1, 2, 4, 8, 16, 32, 64, 128)}

    k = 2
    while k <= _K:
        mk = mlow[k]
        d = k // 2
        while d >= 1:
            vp = _swap(v, d, _R, mlow[d])
            ip = _swap(i, d, _R, mlow[d])
            ge = _gt(v, i, vp, ip)
            take_max = mlow[d] == mk
            takex = ge == take_max
            v = jnp.where(takex, v, vp)
            i = jnp.where(takex, i, ip)
            d //= 2
        k *= 2

    ge = _gt(v[:_K], i[:_K], v[_K:2 * _K], i[_K:2 * _K])
    y1 = jnp.where(ge, v[:_K], v[_K:2 * _K])
    y1i = jnp.where(ge, i[:_K], i[_K:2 * _K])
    ge = _gt(v[2 * _K:3 * _K], i[2 * _K:3 * _K], v[3 * _K:], i[3 * _K:])
    y2 = jnp.where(ge, v[2 * _K:3 * _K], v[3 * _K:])
    y2i = jnp.where(ge, i[2 * _K:3 * _K], i[3 * _K:])
    z = jnp.concatenate([y1, y2], axis=0)
    zi = jnp.concatenate([y1i, y2i], axis=0)
    mblk = mlow[_K][:2 * _K]
    for d in (32, 16, 8, 4, 2, 1):
        md = mlow[d][:2 * _K]
        vp = _swap(z, d, 2 * _K, md)
        ip = _swap(zi, d, 2 * _K, md)
        ge = _gt(z, zi, vp, ip)
        takex = ge == (md == mblk)
        z = jnp.where(takex, z, vp)
        zi = jnp.where(takex, zi, ip)
    ge = _gt(z[:_K], zi[:_K], z[_K:], zi[_K:])
    y = jnp.where(ge, z[:_K], z[_K:])
    yi = jnp.where(ge, zi[:_K], zi[_K:])
    for d in (32, 16, 8, 4, 2, 1):
        md = mlow[d][:_K]
        vp = _swap(y, d, _K, md)
        ip = _swap(yi, d, _K, md)
        ge = _gt(y, yi, vp, ip)
        takex = ge == md
        y = jnp.where(takex, y, vp)
        yi = jnp.where(takex, yi, ip)

    r64 = jax.lax.broadcasted_iota(jnp.int32, (_K, _C), 0)
    m64 = {d: (r64 & d) == 0 for d in (1, 2, 4, 8, 16, 32)}
    w = _C // 2
    while w >= 1:
        a, ai = y[:, :w], yi[:, :w]
        b, bi = y[:, w:2 * w], yi[:, w:2 * w]
        for d in (32, 16, 8, 4, 2, 1):
            md = m64[d][:, :w]
            b = _swap(b, d, _K, md)
            bi = _swap(bi, d, _K, md)
        ge = _gt(a, ai, b, bi)
        y = jnp.where(ge, a, b)
        yi = jnp.where(ge, ai, bi)
        for d in (32, 16, 8, 4, 2, 1):
            md = m64[d][:, :w]
            vp = _swap(y, d, _K, md)
            ip = _swap(yi, d, _K, md)
            ge = _gt(y, yi, vp, ip)
            takex = ge == md
            y = jnp.where(takex, y, vp)
            yi = jnp.where(takex, yi, ip)
        w //= 2

    r6 = jax.lax.broadcasted_iota(jnp.int32, (_K, _K), 0)
    c6 = jax.lax.broadcasted_iota(jnp.int32, (_K, _K), 1)
    vb = jnp.broadcast_to(y, (_K, _K))
    ib = jnp.broadcast_to(yi, (_K, _K))
    vals = jnp.sum(jnp.where(r6 == c6, vb, 0.0), axis=0, keepdims=True)
    idxs = jnp.sum(jnp.where(r6 == c6, ib, 0), axis=0, keepdims=True)
    cs = jnp.sum(jnp.where(r6 <= c6, vb, 0.0), axis=0, keepdims=True)

    j64 = jax.lax.broadcasted_iota(jnp.int32, (1, _K), 1)
    kvec = (j64 + 1).astype(jnp.float32)
    support = (vals - (cs - 1.0) / kvec) > 0.0
    kz = jnp.sum(support.astype(jnp.int32), axis=(0, 1), keepdims=True)
    cs_at = jnp.sum(jnp.where(j64 == kz - 1, cs, 0.0),
                    axis=(0, 1), keepdims=True)
    tau = (cs_at - 1.0) / kz.astype(jnp.float32)

    row0 = x_ref[0:1, :]
    avail = row0 <= tau
    rc = jax.lax.broadcasted_iota(jnp.int32, (_C, _C), 0)
    cc = jax.lax.broadcasted_iota(jnp.int32, (_C, _C), 1)
    ab = jnp.broadcast_to(avail, (_C, _C))
    pc_col = jnp.sum(jnp.where(ab & (cc <= rc), 1, 0), axis=1, keepdims=True)
    av_col = jnp.sum(jnp.where(ab & (cc == rc), 1, 0), axis=1, keepdims=True)
    tgt_col = kz + pc_col - 1
    mfill = (av_col > 0) & (tgt_col == cc)
    fill = jnp.sum(jnp.where(mfill, rc, 0), axis=0, keepdims=True)

    jcol = jax.lax.broadcasted_iota(jnp.int32, (1, _K), 1)
    out = jnp.where(jcol < kz, idxs, fill[:, :_K])
    out_ref[:] = out


def kernel(scores):
    x = scores.reshape(_R, _C)
    out = pl.pallas_call(
        _body,
        out_shape=jax.ShapeDtypeStruct((1, _K), jnp.int32),
    )(x)
    return out.reshape(_K)
